# baseline (speedup 1.0000x reference)
import functools

import jax
import jax.numpy as jnp
from jax import lax
from jax.experimental import pallas as pl
from jax.experimental.pallas import tpu as pltpu
from jax.experimental.pallas import tpu_sc as plsc

_TM = 128


def _route_body(nt, tm, nb, x_ref, nw_ref, wr_ref,
                xn_ref, i0_ref, i1_ref, w0_ref, w1_ref, r0_ref, r1_ref,
                offs_ref, blke_ref, base_ref):
    i = pl.program_id(0)
    bt = x_ref.shape[0]
    e = wr_ref.shape[1]

    @pl.when(i == 0)
    def _():
        base_ref[...] = jnp.zeros_like(base_ref)

    x = x_ref[...]
    ms = jnp.mean(x * x, axis=-1, keepdims=True)
    xn = x * nw_ref[...] * jax.lax.rsqrt(ms + 1e-6)
    xn_ref[...] = xn

    aff = jnp.dot(xn, wr_ref[...], preferred_element_type=jnp.float32)
    iota = jax.lax.broadcasted_iota(jnp.int32, aff.shape, 1)
    m0 = jnp.max(aff, axis=-1, keepdims=True)
    i0 = jnp.min(jnp.where(aff == m0, iota, e), axis=-1, keepdims=True)
    aff1 = jnp.where(iota == i0, -jnp.inf, aff)
    m1 = jnp.max(aff1, axis=-1, keepdims=True)
    i1 = jnp.min(jnp.where(aff1 == m1, iota, e), axis=-1, keepdims=True)
    i0_ref[...] = i0
    i1_ref[...] = i1
    w0_ref[...] = m0
    w1_ref[...] = m1

    oh0 = (iota == i0).astype(jnp.float32)
    oh1 = (iota == i1).astype(jnp.float32)
    ri = jax.lax.broadcasted_iota(jnp.int32, (bt, bt), 0)
    ci = jax.lax.broadcasted_iota(jnp.int32, (bt, bt), 1)
    ls = (ri > ci).astype(jnp.bfloat16)
    c0 = jnp.dot(ls, oh0.astype(jnp.bfloat16), preferred_element_type=jnp.float32)
    c1 = jnp.dot(ls, oh1.astype(jnp.bfloat16), preferred_element_type=jnp.float32)
    base = base_ref[...]
    s0 = jnp.sum(oh0, axis=0, keepdims=True)
    s1 = jnp.sum(oh1, axis=0, keepdims=True)
    r0 = jnp.sum(oh0 * (c0 + base), axis=-1, keepdims=True)
    r1 = jnp.sum(oh1 * (c1 + base + s0), axis=-1, keepdims=True)
    r0_ref[...] = r0.astype(jnp.int32)
    r1_ref[...] = r1.astype(jnp.int32)
    total = base + s0 + s1
    base_ref[...] = total

    padded = jnp.floor((total + (tm - 1)) * (1.0 / tm)) * tm
    re = jax.lax.broadcasted_iota(jnp.int32, (e, e), 0)
    ce = jax.lax.broadcasted_iota(jnp.int32, (e, e), 1)
    lse = (re < ce).astype(jnp.float32)
    offs = jnp.dot(padded, lse, preferred_element_type=jnp.float32)
    ends = offs + padded
    btm = (jax.lax.broadcasted_iota(jnp.int32, (e, nb), 1)
           .astype(jnp.float32) * tm)
    blke = jnp.sum((ends.reshape(e, 1) <= btm).astype(jnp.int32), axis=0,
                   keepdims=True)
    offs_ref[...] = offs.astype(jnp.int32)
    blke_ref[...] = jnp.minimum(blke, e - 1)


def _pos_body(i0_ref, i1_ref, r0_ref, r1_ref, offs_ref, p0_ref, p1_ref):
    offs = offs_ref[...]
    e = offs.shape[1]
    bt = i0_ref.shape[0]
    iota = jax.lax.broadcasted_iota(jnp.int32, (bt, e), 1)
    oh0 = (i0_ref[...] == iota).astype(jnp.int32)
    oh1 = (i1_ref[...] == iota).astype(jnp.int32)
    p0_ref[...] = jnp.sum(oh0 * offs, axis=-1, keepdims=True) + r0_ref[...]
    p1_ref[...] = jnp.sum(oh1 * offs, axis=-1, keepdims=True) + r1_ref[...]


def _ffn_body(blke_ref, xg_ref, w1_ref, wg_ref, w2_ref, b1_ref, b2_ref,
              yg_ref):
    xb = xg_ref[...].astype(jnp.bfloat16)
    h = jnp.dot(xb, w1_ref[0], preferred_element_type=jnp.float32)
    h = h + b1_ref[0]
    g = jnp.dot(h.astype(jnp.bfloat16), wg_ref[...],
                preferred_element_type=jnp.float32)
    h = h * (g * (1.0 / (1.0 + jnp.exp(-g))))
    y = jnp.dot(h.astype(jnp.bfloat16), w2_ref[0],
                preferred_element_type=jnp.float32)
    yg_ref[...] = y + b2_ref[0]


def _combine_body(x_ref, nw_ref, r0_ref, r1_ref, w0_ref, w1_ref,
                  w1s_ref, wgs_ref, w2s_ref, b1s_ref, b2s_ref, out_ref):
    x = x_ref[...]
    ms = jnp.mean(x * x, axis=-1, keepdims=True)
    xn = x * nw_ref[...] * jax.lax.rsqrt(ms + 1e-6)
    h = jnp.dot(xn.astype(jnp.bfloat16), w1s_ref[...],
                preferred_element_type=jnp.float32)
    h = h + b1s_ref[...]
    g = jnp.dot(h.astype(jnp.bfloat16), wgs_ref[...],
                preferred_element_type=jnp.float32)
    h = h * (g * (1.0 / (1.0 + jnp.exp(-g))))
    y = jnp.dot(h.astype(jnp.bfloat16), w2s_ref[...],
                preferred_element_type=jnp.float32)
    y = y + b2s_ref[...]
    out_ref[...] = x + y + w0_ref[...] * r0_ref[...] + w1_ref[...] * r1_ref[...]


def _make_scatter(T, D, P):
    info = plsc.get_sparse_core_info()
    nc, ns = info.num_cores, info.num_subcores
    nw = nc * ns
    chunk = T // nw
    half = chunk // 2
    mesh = plsc.VectorSubcoreMesh(core_axis_name="c", subcore_axis_name="s")

    @functools.partial(
        pl.kernel, mesh=mesh,
        out_type=[
            jax.ShapeDtypeStruct((P, D), jnp.float32),
        ],
        scratch_types=[
            pltpu.VMEM((half, D), jnp.float32),
            pltpu.VMEM((chunk,), jnp.int32),
            pltpu.VMEM((chunk,), jnp.int32),
            pltpu.VMEM((half,), jnp.int32),
            pltpu.SemaphoreType.DMA,
        ],
    )
    def scatter_k(xn_hbm, pos0_hbm, pos1_hbm, xg_hbm,
                  rows_v, p0_v, p1_v, pb, sem):
        wid = lax.axis_index("s") * nc + lax.axis_index("c")
        base = wid * chunk
        pltpu.sync_copy(pos0_hbm.at[pl.ds(base, chunk)], p0_v)
        pltpu.sync_copy(pos1_hbm.at[pl.ds(base, chunk)], p1_v)
        for h in range(2):
            pltpu.sync_copy(xn_hbm.at[pl.ds(base + h * half, half)], rows_v)
            for k in range(2):
                pv = p0_v if k == 0 else p1_v
                for j in range(half // 16):
                    pb[pl.ds(j * 16, 16)] = pv[pl.ds(h * half + j * 16, 16)]
                pltpu.async_copy(rows_v, xg_hbm.at[pb], sem).wait()

    return scatter_k


def _make_gather(T, D, P):
    info = plsc.get_sparse_core_info()
    nc, ns = info.num_cores, info.num_subcores
    nw = nc * ns
    chunk = T // nw
    half = chunk // 2
    mesh = plsc.VectorSubcoreMesh(core_axis_name="c", subcore_axis_name="s")

    @functools.partial(
        pl.kernel, mesh=mesh,
        out_type=[
            jax.ShapeDtypeStruct((T, D), jnp.float32),
            jax.ShapeDtypeStruct((T, D), jnp.float32),
        ],
        scratch_types=[
            pltpu.VMEM((half, D), jnp.float32),
            pltpu.VMEM((chunk,), jnp.int32),
            pltpu.VMEM((chunk,), jnp.int32),
            pltpu.VMEM((half,), jnp.int32),
            pltpu.SemaphoreType.DMA,
        ],
    )
    def gather_k(yg_hbm, pos0_hbm, pos1_hbm, r0_hbm, r1_hbm,
                 rows_v, p0_v, p1_v, pb, sem):
        wid = lax.axis_index("s") * nc + lax.axis_index("c")
        base = wid * chunk
        pltpu.sync_copy(pos0_hbm.at[pl.ds(base, chunk)], p0_v)
        pltpu.sync_copy(pos1_hbm.at[pl.ds(base, chunk)], p1_v)
        for k in range(2):
            pv = p0_v if k == 0 else p1_v
            dst = r0_hbm if k == 0 else r1_hbm
            for h in range(2):
                for j in range(half // 16):
                    pb[pl.ds(j * 16, 16)] = pv[pl.ds(h * half + j * 16, 16)]
                pltpu.async_copy(yg_hbm.at[pb], rows_v, sem).wait()
                pltpu.sync_copy(rows_v, dst.at[pl.ds(base + h * half, half)])

    return gather_k


def kernel(x, norm_w, Wr, W1s, b1s, W2s, b2s, Wgs, W1r, b1r, W2r, b2r, Wgr):
    B, S, D = x.shape
    E = Wr.shape[-1]
    H = W1r.shape[-1]
    T = B * S
    TM = _TM
    P = T * 2 + E * TM
    NB = P // TM
    BTA = 512

    xf = x.reshape(T, D)
    nw = norm_w.reshape(1, D)
    bf = jnp.bfloat16

    nt = T // BTA
    outs = pl.pallas_call(
        functools.partial(_route_body, nt, TM, NB),
        grid=(nt,),
        in_specs=[
            pl.BlockSpec((BTA, D), lambda i: (i, 0)),
            pl.BlockSpec((1, D), lambda i: (0, 0)),
            pl.BlockSpec((D, E), lambda i: (0, 0)),
        ],
        out_specs=[
            pl.BlockSpec((BTA, D), lambda i: (i, 0)),
            pl.BlockSpec((BTA, 1), lambda i: (i, 0)),
            pl.BlockSpec((BTA, 1), lambda i: (i, 0)),
            pl.BlockSpec((BTA, 1), lambda i: (i, 0)),
            pl.BlockSpec((BTA, 1), lambda i: (i, 0)),
            pl.BlockSpec((BTA, 1), lambda i: (i, 0)),
            pl.BlockSpec((BTA, 1), lambda i: (i, 0)),
            pl.BlockSpec((1, E), lambda i: (0, 0)),
            pl.BlockSpec((1, NB), lambda i: (0, 0)),
        ],
        out_shape=[
            jax.ShapeDtypeStruct((T, D), jnp.float32),
            jax.ShapeDtypeStruct((T, 1), jnp.int32),
            jax.ShapeDtypeStruct((T, 1), jnp.int32),
            jax.ShapeDtypeStruct((T, 1), jnp.float32),
            jax.ShapeDtypeStruct((T, 1), jnp.float32),
            jax.ShapeDtypeStruct((T, 1), jnp.int32),
            jax.ShapeDtypeStruct((T, 1), jnp.int32),
            jax.ShapeDtypeStruct((1, E), jnp.int32),
            jax.ShapeDtypeStruct((1, NB), jnp.int32),
        ],
        scratch_shapes=[pltpu.VMEM((1, E), jnp.float32)],
    )(xf, nw, Wr)
    xn, i0, i1, w0, w1, r0, r1, offs, blke = outs

    BTP = 512
    pos0, pos1 = pl.pallas_call(
        _pos_body,
        grid=(T // BTP,),
        in_specs=[
            pl.BlockSpec((BTP, 1), lambda i: (i, 0)),
            pl.BlockSpec((BTP, 1), lambda i: (i, 0)),
            pl.BlockSpec((BTP, 1), lambda i: (i, 0)),
            pl.BlockSpec((BTP, 1), lambda i: (i, 0)),
            pl.BlockSpec((1, E), lambda i: (0, 0)),
        ],
        out_specs=[
            pl.BlockSpec((BTP, 1), lambda i: (i, 0)),
            pl.BlockSpec((BTP, 1), lambda i: (i, 0)),
        ],
        out_shape=[
            jax.ShapeDtypeStruct((T, 1), jnp.int32),
            jax.ShapeDtypeStruct((T, 1), jnp.int32),
        ],
    )(i0, i1, r0, r1, offs)
    pos0 = pos0.reshape(T)
    pos1 = pos1.reshape(T)

    (xg,) = _make_scatter(T, D, P)(xn, pos0, pos1)

    w1r_b = W1r.astype(bf)
    w2r_b = W2r.astype(bf)
    wgr_b = Wgr.astype(bf)
    yg = pl.pallas_call(
        _ffn_body,
        grid_spec=pltpu.PrefetchScalarGridSpec(
            num_scalar_prefetch=1,
            grid=(NB,),
            in_specs=[
                pl.BlockSpec((TM, D), lambda b, be: (b, 0)),
                pl.BlockSpec((1, D, H), lambda b, be: (be[b], 0, 0)),
                pl.BlockSpec((H, H), lambda b, be: (0, 0)),
                pl.BlockSpec((1, H, D), lambda b, be: (be[b], 0, 0)),
                pl.BlockSpec((1, 1, H), lambda b, be: (be[b], 0, 0)),
                pl.BlockSpec((1, 1, D), lambda b, be: (be[b], 0, 0)),
            ],
            out_specs=pl.BlockSpec((TM, D), lambda b, be: (b, 0)),
        ),
        out_shape=jax.ShapeDtypeStruct((P, D), jnp.float32),
    )(blke.reshape(NB), xg, w1r_b, wgr_b, w2r_b, b1r, b2r)

    rg0, rg1 = _make_gather(T, D, P)(yg, pos0, pos1)

    BTC = 512
    out = pl.pallas_call(
        _combine_body,
        grid=(T // BTC,),
        in_specs=[
            pl.BlockSpec((BTC, D), lambda i: (i, 0)),
            pl.BlockSpec((1, D), lambda i: (0, 0)),
            pl.BlockSpec((BTC, D), lambda i: (i, 0)),
            pl.BlockSpec((BTC, D), lambda i: (i, 0)),
            pl.BlockSpec((BTC, 1), lambda i: (i, 0)),
            pl.BlockSpec((BTC, 1), lambda i: (i, 0)),
            pl.BlockSpec((D, H), lambda i: (0, 0)),
            pl.BlockSpec((H, H), lambda i: (0, 0)),
            pl.BlockSpec((H, D), lambda i: (0, 0)),
            pl.BlockSpec((1, H), lambda i: (0, 0)),
            pl.BlockSpec((1, D), lambda i: (0, 0)),
        ],
        out_specs=pl.BlockSpec((BTC, D), lambda i: (i, 0)),
        out_shape=jax.ShapeDtypeStruct((T, D), jnp.float32),
    )(xf, nw, rg0, rg1, w0, w1,
      W1s[0].astype(bf), Wgs.astype(bf), W2s[0].astype(bf),
      b1s[:, 0, :], b2s[:, 0, :])
    return out.reshape(B, S, D)


# speedup vs baseline: 1.0012x; 1.0012x over previous
import functools

import jax
import jax.numpy as jnp
from jax import lax
from jax.experimental import pallas as pl
from jax.experimental.pallas import tpu as pltpu
from jax.experimental.pallas import tpu_sc as plsc

_TM = 128


def _route_body(nt, tm, nb, cb, x_ref, nw_ref, wr_ref,
                xn_ref, i0_ref, i1_ref, w0_ref, w1_ref, r0_ref, r1_ref,
                blke_ref, blkrow_ref, base_ref):
    i = pl.program_id(0)
    bt = x_ref.shape[0]
    e = wr_ref.shape[1]

    @pl.when(i == 0)
    def _():
        base_ref[...] = jnp.zeros_like(base_ref)

    x = x_ref[...]
    ms = jnp.mean(x * x, axis=-1, keepdims=True)
    xn = x * nw_ref[...] * jax.lax.rsqrt(ms + 1e-6)
    xn_ref[...] = xn

    aff = jnp.dot(xn, wr_ref[...], preferred_element_type=jnp.float32)
    iota = jax.lax.broadcasted_iota(jnp.int32, aff.shape, 1)
    m0 = jnp.max(aff, axis=-1, keepdims=True)
    i0 = jnp.min(jnp.where(aff == m0, iota, e), axis=-1, keepdims=True)
    aff1 = jnp.where(iota == i0, -jnp.inf, aff)
    m1 = jnp.max(aff1, axis=-1, keepdims=True)
    i1 = jnp.min(jnp.where(aff1 == m1, iota, e), axis=-1, keepdims=True)
    i0_ref[...] = i0
    i1_ref[...] = i1
    w0_ref[...] = m0
    w1_ref[...] = m1

    oh0 = (iota == i0).astype(jnp.float32)
    oh1 = (iota == i1).astype(jnp.float32)
    ri = jax.lax.broadcasted_iota(jnp.int32, (bt, bt), 0)
    ci = jax.lax.broadcasted_iota(jnp.int32, (bt, bt), 1)
    ls = (ri > ci).astype(jnp.bfloat16)
    c0 = jnp.dot(ls, oh0.astype(jnp.bfloat16), preferred_element_type=jnp.float32)
    c1 = jnp.dot(ls, oh1.astype(jnp.bfloat16), preferred_element_type=jnp.float32)
    base = base_ref[...]
    s0 = jnp.sum(oh0, axis=0, keepdims=True)
    s1 = jnp.sum(oh1, axis=0, keepdims=True)
    r0 = jnp.sum(oh0 * (c0 + base), axis=-1, keepdims=True)
    r1 = jnp.sum(oh1 * (c1 + base + s0), axis=-1, keepdims=True)
    r0_ref[...] = r0.astype(jnp.int32)
    r1_ref[...] = r1.astype(jnp.int32)
    total = base + s0 + s1
    base_ref[...] = total

    nblk = jnp.floor((total + (tm - 1)) * (1.0 / tm))
    re = jax.lax.broadcasted_iota(jnp.int32, (e, e), 0)
    ce = jax.lax.broadcasted_iota(jnp.int32, (e, e), 1)
    lse = (re < ce).astype(jnp.float32)
    start = jnp.dot(nblk, lse, preferred_element_type=jnp.float32)
    ends = start + nblk
    ntot = jnp.sum(nblk, axis=-1, keepdims=True)
    bi = (jax.lax.broadcasted_iota(jnp.int32, (e, nb), 1)
          .astype(jnp.float32))
    eofb = jnp.sum((ends.reshape(e, 1) <= bi).astype(jnp.int32), axis=0,
                   keepdims=True)
    eofb = jnp.minimum(eofb, e - 1)
    ohb = (eofb == jax.lax.broadcasted_iota(jnp.int32, (e, nb), 0))
    startofb = jnp.sum(jnp.where(ohb, start.reshape(e, 1), 0.0), axis=0,
                       keepdims=True)
    bi0 = bi[0:1, :]
    row = (eofb * cb).astype(jnp.float32) + bi0 - startofb
    row = jnp.where(bi0 < ntot, row, float(e * cb))
    blke_ref[...] = eofb
    blkrow_ref[...] = row.astype(jnp.int32)


def _ffn_body(blke_ref, blkrow_ref, xg_ref, w1_ref, wg_ref, w2_ref,
              b1_ref, b2_ref, yg_ref):
    xb = xg_ref[...].astype(jnp.bfloat16)
    h = jnp.dot(xb, w1_ref[0], preferred_element_type=jnp.float32)
    h = h + b1_ref[0]
    g = jnp.dot(h.astype(jnp.bfloat16), wg_ref[...],
                preferred_element_type=jnp.float32)
    h = h * (g * (1.0 / (1.0 + jnp.exp(-g))))
    y = jnp.dot(h.astype(jnp.bfloat16), w2_ref[0],
                preferred_element_type=jnp.float32)
    yg_ref[...] = y + b2_ref[0]


def _combine_body(x_ref, nw_ref, r0_ref, r1_ref, w0_ref, w1_ref,
                  w1s_ref, wgs_ref, w2s_ref, b1s_ref, b2s_ref, out_ref):
    x = x_ref[...]
    ms = jnp.mean(x * x, axis=-1, keepdims=True)
    xn = x * nw_ref[...] * jax.lax.rsqrt(ms + 1e-6)
    h = jnp.dot(xn.astype(jnp.bfloat16), w1s_ref[...],
                preferred_element_type=jnp.float32)
    h = h + b1s_ref[...]
    g = jnp.dot(h.astype(jnp.bfloat16), wgs_ref[...],
                preferred_element_type=jnp.float32)
    h = h * (g * (1.0 / (1.0 + jnp.exp(-g))))
    y = jnp.dot(h.astype(jnp.bfloat16), w2s_ref[...],
                preferred_element_type=jnp.float32)
    y = y + b2s_ref[...]
    out_ref[...] = (x + y
                    + w0_ref[...] * r0_ref[...] + w1_ref[...] * r1_ref[...])


def _sc_dims(T):
    info = plsc.get_sparse_core_info()
    nc, ns = info.num_cores, info.num_subcores
    return nc, ns, T // (nc * ns)


def _make_scatter(T, D, P2, C):
    nc, ns, chunk = _sc_dims(T)
    half = chunk // 2
    mesh = plsc.VectorSubcoreMesh(core_axis_name="c", subcore_axis_name="s")

    q = chunk // 4

    @functools.partial(
        pl.kernel, mesh=mesh,
        out_type=[jax.ShapeDtypeStruct((P2, D), jnp.float32)],
        scratch_types=[
            pltpu.VMEM((q, D), jnp.float32),
            pltpu.VMEM((q, D), jnp.float32),
            pltpu.VMEM((chunk,), jnp.int32),
            pltpu.VMEM((chunk,), jnp.int32),
            pltpu.VMEM((chunk,), jnp.int32),
            pltpu.VMEM((chunk,), jnp.int32),
            pltpu.VMEM((q,), jnp.int32),
            pltpu.VMEM((q,), jnp.int32),
            pltpu.VMEM((q,), jnp.int32),
            pltpu.VMEM((q,), jnp.int32),
            pltpu.SemaphoreType.DMA,
            pltpu.SemaphoreType.DMA,
            pltpu.SemaphoreType.DMA,
            pltpu.SemaphoreType.DMA,
            pltpu.SemaphoreType.DMA,
            pltpu.SemaphoreType.DMA,
        ],
    )
    def scatter_k(xn_hbm, i0_hbm, i1_hbm, r0_hbm, r1_hbm, xg_hbm,
                  rows_a, rows_b, i0_v, i1_v, r0_v, r1_v,
                  pa0, pa1, pb0, pb1, la, lb, sa0, sa1, sb0, sb1):
        wid = lax.axis_index("s") * nc + lax.axis_index("c")
        base = wid * chunk
        pltpu.sync_copy(i0_hbm.at[pl.ds(base, chunk)], i0_v)
        pltpu.sync_copy(i1_hbm.at[pl.ds(base, chunk)], i1_v)
        pltpu.sync_copy(r0_hbm.at[pl.ds(base, chunk)], r0_v)
        pltpu.sync_copy(r1_hbm.at[pl.ds(base, chunk)], r1_v)

        rows = (rows_a, rows_b)
        idx = ((pa0, pa1), (pb0, pb1))
        lsem = (la, lb)
        ssem = ((sa0, sa1), (sb0, sb1))

        def fill(pb, iv, rv, t):
            for j in range(q // 16):
                sl = pl.ds(t * q + j * 16, 16)
                pb[pl.ds(j * 16, 16)] = iv[sl] * C + rv[sl]

        lds = [None, None]
        scs = [None, None]
        lds[0] = pltpu.async_copy(xn_hbm.at[pl.ds(base, q)], rows[0], la)
        lds[1] = pltpu.async_copy(xn_hbm.at[pl.ds(base + q, q)], rows[1], lb)
        for t in range(4):
            par = t % 2
            fill(idx[par][0], i0_v, r0_v, t)
            fill(idx[par][1], i1_v, r1_v, t)
            lds[par].wait()
            s0 = pltpu.async_copy(rows[par], xg_hbm.at[idx[par][0]],
                                  ssem[par][0])
            s1 = pltpu.async_copy(rows[par], xg_hbm.at[idx[par][1]],
                                  ssem[par][1])
            scs[par] = (s0, s1)
            if 1 <= t <= 2:
                opar = 1 - par
                scs[opar][0].wait()
                scs[opar][1].wait()
                lds[opar] = pltpu.async_copy(
                    xn_hbm.at[pl.ds(base + (t + 1) * q, q)], rows[opar],
                    lsem[opar])
        scs[0][0].wait()
        scs[0][1].wait()
        scs[1][0].wait()
        scs[1][1].wait()

    return scatter_k


def _make_gather(T, D, P2, C):
    nc, ns, chunk = _sc_dims(T)
    half = chunk // 2
    mesh = plsc.VectorSubcoreMesh(core_axis_name="c", subcore_axis_name="s")

    q = chunk // 4

    @functools.partial(
        pl.kernel, mesh=mesh,
        out_type=[
            jax.ShapeDtypeStruct((T, D), jnp.float32),
            jax.ShapeDtypeStruct((T, D), jnp.float32),
        ],
        scratch_types=[
            pltpu.VMEM((q, D), jnp.float32),
            pltpu.VMEM((q, D), jnp.float32),
            pltpu.VMEM((chunk,), jnp.int32),
            pltpu.VMEM((chunk,), jnp.int32),
            pltpu.VMEM((chunk,), jnp.int32),
            pltpu.VMEM((chunk,), jnp.int32),
            pltpu.VMEM((q,), jnp.int32),
            pltpu.VMEM((q,), jnp.int32),
            pltpu.SemaphoreType.DMA,
            pltpu.SemaphoreType.DMA,
            pltpu.SemaphoreType.DMA,
            pltpu.SemaphoreType.DMA,
        ],
    )
    def gather_k(yg_hbm, i0_hbm, i1_hbm, r0_hbm, r1_hbm,
                 r0out_hbm, r1out_hbm,
                 rows_a, rows_b, i0_v, i1_v, r0_v, r1_v,
                 pb0, pb1, ga, gb, wa, wb):
        wid = lax.axis_index("s") * nc + lax.axis_index("c")
        base = wid * chunk
        pltpu.sync_copy(i0_hbm.at[pl.ds(base, chunk)], i0_v)
        pltpu.sync_copy(i1_hbm.at[pl.ds(base, chunk)], i1_v)
        pltpu.sync_copy(r0_hbm.at[pl.ds(base, chunk)], r0_v)
        pltpu.sync_copy(r1_hbm.at[pl.ds(base, chunk)], r1_v)

        def fill(pb, iv, rv, t):
            for j in range(q // 16):
                sl = pl.ds(t * q + j * 16, 16)
                pb[pl.ds(j * 16, 16)] = iv[sl] * C + rv[sl]

        pw0 = pw1 = None
        for t in range(4):
            fill(pb0, i0_v, r0_v, t)
            fill(pb1, i1_v, r1_v, t)
            if pw0 is not None:
                pw0.wait()
                pw1.wait()
            g0 = pltpu.async_copy(yg_hbm.at[pb0], rows_a, ga)
            g1 = pltpu.async_copy(yg_hbm.at[pb1], rows_b, gb)
            g0.wait()
            pw0 = pltpu.async_copy(rows_a,
                                   r0out_hbm.at[pl.ds(base + t * q, q)], wa)
            g1.wait()
            pw1 = pltpu.async_copy(rows_b,
                                   r1out_hbm.at[pl.ds(base + t * q, q)], wb)
        pw0.wait()
        pw1.wait()

    return gather_k


def kernel(x, norm_w, Wr, W1s, b1s, W2s, b2s, Wgs, W1r, b1r, W2r, b2r, Wgr):
    B, S, D = x.shape
    E = Wr.shape[-1]
    H = W1r.shape[-1]
    T = B * S
    TM = _TM
    CAP = T
    CB = CAP // TM
    P2 = E * CAP + TM
    NB = (2 * T + E * TM) // TM
    BTA = 512

    xf = x.reshape(T, D)
    nw = norm_w.reshape(1, D)
    bf = jnp.bfloat16

    nt = T // BTA
    outs = pl.pallas_call(
        functools.partial(_route_body, nt, TM, NB, CB),
        grid=(nt,),
        in_specs=[
            pl.BlockSpec((BTA, D), lambda i: (i, 0)),
            pl.BlockSpec((1, D), lambda i: (0, 0)),
            pl.BlockSpec((D, E), lambda i: (0, 0)),
        ],
        out_specs=[
            pl.BlockSpec((BTA, D), lambda i: (i, 0)),
            pl.BlockSpec((BTA, 1), lambda i: (i, 0)),
            pl.BlockSpec((BTA, 1), lambda i: (i, 0)),
            pl.BlockSpec((BTA, 1), lambda i: (i, 0)),
            pl.BlockSpec((BTA, 1), lambda i: (i, 0)),
            pl.BlockSpec((BTA, 1), lambda i: (i, 0)),
            pl.BlockSpec((BTA, 1), lambda i: (i, 0)),
            pl.BlockSpec((1, NB), lambda i: (0, 0)),
            pl.BlockSpec((1, NB), lambda i: (0, 0)),
        ],
        out_shape=[
            jax.ShapeDtypeStruct((T, D), jnp.float32),
            jax.ShapeDtypeStruct((T, 1), jnp.int32),
            jax.ShapeDtypeStruct((T, 1), jnp.int32),
            jax.ShapeDtypeStruct((T, 1), jnp.float32),
            jax.ShapeDtypeStruct((T, 1), jnp.float32),
            jax.ShapeDtypeStruct((T, 1), jnp.int32),
            jax.ShapeDtypeStruct((T, 1), jnp.int32),
            jax.ShapeDtypeStruct((1, NB), jnp.int32),
            jax.ShapeDtypeStruct((1, NB), jnp.int32),
        ],
        scratch_shapes=[pltpu.VMEM((1, E), jnp.float32)],
    )(xf, nw, Wr)
    xn, i0, i1, w0, w1, r0, r1, blke, blkrow = outs
    i0f = i0.reshape(T)
    i1f = i1.reshape(T)
    r0f = r0.reshape(T)
    r1f = r1.reshape(T)

    (xg,) = _make_scatter(T, D, P2, CAP)(xn, i0f, i1f, r0f, r1f)

    w1r_b = W1r.astype(bf)
    w2r_b = W2r.astype(bf)
    wgr_b = Wgr.astype(bf)
    yg = pl.pallas_call(
        _ffn_body,
        grid_spec=pltpu.PrefetchScalarGridSpec(
            num_scalar_prefetch=2,
            grid=(NB,),
            in_specs=[
                pl.BlockSpec((TM, D), lambda b, be, br: (br[b], 0)),
                pl.BlockSpec((1, D, H), lambda b, be, br: (be[b], 0, 0)),
                pl.BlockSpec((H, H), lambda b, be, br: (0, 0)),
                pl.BlockSpec((1, H, D), lambda b, be, br: (be[b], 0, 0)),
                pl.BlockSpec((1, 1, H), lambda b, be, br: (be[b], 0, 0)),
                pl.BlockSpec((1, 1, D), lambda b, be, br: (be[b], 0, 0)),
            ],
            out_specs=pl.BlockSpec((TM, D), lambda b, be, br: (br[b], 0)),
        ),
        out_shape=jax.ShapeDtypeStruct((P2, D), jnp.float32),
    )(blke.reshape(NB), blkrow.reshape(NB), xg, w1r_b, wgr_b, w2r_b,
      b1r, b2r)

    rg0, rg1 = _make_gather(T, D, P2, CAP)(yg, i0f, i1f, r0f, r1f)

    BTC = 512
    out = pl.pallas_call(
        _combine_body,
        grid=(T // BTC,),
        in_specs=[
            pl.BlockSpec((BTC, D), lambda i: (i, 0)),
            pl.BlockSpec((1, D), lambda i: (0, 0)),
            pl.BlockSpec((BTC, D), lambda i: (i, 0)),
            pl.BlockSpec((BTC, D), lambda i: (i, 0)),
            pl.BlockSpec((BTC, 1), lambda i: (i, 0)),
            pl.BlockSpec((BTC, 1), lambda i: (i, 0)),
            pl.BlockSpec((D, H), lambda i: (0, 0)),
            pl.BlockSpec((H, H), lambda i: (0, 0)),
            pl.BlockSpec((H, D), lambda i: (0, 0)),
            pl.BlockSpec((1, H), lambda i: (0, 0)),
            pl.BlockSpec((1, D), lambda i: (0, 0)),
        ],
        out_specs=pl.BlockSpec((BTC, D), lambda i: (i, 0)),
        out_shape=jax.ShapeDtypeStruct((T, D), jnp.float32),
    )(xf, nw, rg0, rg1, w0, w1,
      W1s[0].astype(bf), Wgs.astype(bf), W2s[0].astype(bf),
      b1s[:, 0, :], b2s[:, 0, :])
    return out.reshape(B, S, D)


# speedup vs baseline: 1.1036x; 1.1023x over previous
import functools

import jax
import jax.numpy as jnp
from jax import lax
from jax.experimental import pallas as pl
from jax.experimental.pallas import tpu as pltpu
from jax.experimental.pallas import tpu_sc as plsc

_TM = 256


def _route_body(nt, tm, nb, cb, x_ref, nw_ref, wr_ref,
                xn_ref, i0_ref, i1_ref, w0_ref, w1_ref, r0_ref, r1_ref,
                blke_ref, blkrow_ref, base_ref):
    i = pl.program_id(0)
    bt = x_ref.shape[0]
    e = wr_ref.shape[1]

    @pl.when(i == 0)
    def _():
        base_ref[...] = jnp.zeros_like(base_ref)

    x = x_ref[...]
    ms = jnp.mean(x * x, axis=-1, keepdims=True)
    xn = x * nw_ref[...] * jax.lax.rsqrt(ms + 1e-6)
    xn_ref[...] = xn

    aff = jnp.dot(xn, wr_ref[...], preferred_element_type=jnp.float32)
    iota = jax.lax.broadcasted_iota(jnp.int32, aff.shape, 1)
    m0 = jnp.max(aff, axis=-1, keepdims=True)
    i0 = jnp.min(jnp.where(aff == m0, iota, e), axis=-1, keepdims=True)
    aff1 = jnp.where(iota == i0, -jnp.inf, aff)
    m1 = jnp.max(aff1, axis=-1, keepdims=True)
    i1 = jnp.min(jnp.where(aff1 == m1, iota, e), axis=-1, keepdims=True)
    i0_ref[...] = i0[:, 0]
    i1_ref[...] = i1[:, 0]
    w0_ref[...] = m0
    w1_ref[...] = m1

    oh0 = (iota == i0).astype(jnp.float32)
    oh1 = (iota == i1).astype(jnp.float32)
    ri = jax.lax.broadcasted_iota(jnp.int32, (bt, bt), 0)
    ci = jax.lax.broadcasted_iota(jnp.int32, (bt, bt), 1)
    ls = (ri > ci).astype(jnp.bfloat16)
    c0 = jnp.dot(ls, oh0.astype(jnp.bfloat16), preferred_element_type=jnp.float32)
    c1 = jnp.dot(ls, oh1.astype(jnp.bfloat16), preferred_element_type=jnp.float32)
    base = base_ref[...]
    s0 = jnp.sum(oh0, axis=0, keepdims=True)
    s1 = jnp.sum(oh1, axis=0, keepdims=True)
    r0 = jnp.sum(oh0 * (c0 + base), axis=-1, keepdims=True)
    r1 = jnp.sum(oh1 * (c1 + base + s0), axis=-1, keepdims=True)
    r0_ref[...] = r0.astype(jnp.int32)[:, 0]
    r1_ref[...] = r1.astype(jnp.int32)[:, 0]
    total = base + s0 + s1
    base_ref[...] = total

    nblk = jnp.floor((total + (tm - 1)) * (1.0 / tm))
    re = jax.lax.broadcasted_iota(jnp.int32, (e, e), 0)
    ce = jax.lax.broadcasted_iota(jnp.int32, (e, e), 1)
    lse = (re < ce).astype(jnp.float32)
    start = jnp.dot(nblk, lse, preferred_element_type=jnp.float32)
    ends = start + nblk
    ntot = jnp.sum(nblk, axis=-1, keepdims=True)
    bi = (jax.lax.broadcasted_iota(jnp.int32, (e, nb), 1)
          .astype(jnp.float32))
    eofb = jnp.sum((ends.reshape(e, 1) <= bi).astype(jnp.int32), axis=0,
                   keepdims=True)
    eofb = jnp.minimum(eofb, e - 1)
    ohb = (eofb == jax.lax.broadcasted_iota(jnp.int32, (e, nb), 0))
    startofb = jnp.sum(jnp.where(ohb, start.reshape(e, 1), 0.0), axis=0,
                       keepdims=True)
    bi0 = bi[0:1, :]
    row = (eofb * cb).astype(jnp.float32) + bi0 - startofb
    row = jnp.where(bi0 < ntot, row, float(e * cb))
    blke_ref[...] = eofb
    blkrow_ref[...] = row.astype(jnp.int32)


def _ffn_body(blke_ref, blkrow_ref, xg_ref, w1_ref, wg_ref, w2_ref,
              b1_ref, b2_ref, yg_ref):
    xb = xg_ref[...].astype(jnp.bfloat16)
    h = jnp.dot(xb, w1_ref[0], preferred_element_type=jnp.float32)
    h = h + b1_ref[0]
    g = jnp.dot(h.astype(jnp.bfloat16), wg_ref[...],
                preferred_element_type=jnp.float32)
    h = h * (g * (1.0 / (1.0 + jnp.exp(-g))))
    y = jnp.dot(h.astype(jnp.bfloat16), w2_ref[0],
                preferred_element_type=jnp.float32)
    yg_ref[...] = y + b2_ref[0]


def _combine_body(x_ref, nw_ref, r0_ref, r1_ref, w0_ref, w1_ref,
                  w1s_ref, wgs_ref, w2s_ref, b1s_ref, b2s_ref, out_ref):
    x = x_ref[...]
    ms = jnp.mean(x * x, axis=-1, keepdims=True)
    xn = x * nw_ref[...] * jax.lax.rsqrt(ms + 1e-6)
    h = jnp.dot(xn.astype(jnp.bfloat16), w1s_ref[...],
                preferred_element_type=jnp.float32)
    h = h + b1s_ref[...]
    g = jnp.dot(h.astype(jnp.bfloat16), wgs_ref[...],
                preferred_element_type=jnp.float32)
    h = h * (g * (1.0 / (1.0 + jnp.exp(-g))))
    y = jnp.dot(h.astype(jnp.bfloat16), w2s_ref[...],
                preferred_element_type=jnp.float32)
    y = y + b2s_ref[...]
    out_ref[...] = (x + y
                    + w0_ref[...] * r0_ref[...] + w1_ref[...] * r1_ref[...])


def _sc_dims(T):
    info = plsc.get_sparse_core_info()
    nc, ns = info.num_cores, info.num_subcores
    return nc, ns, T // (nc * ns)


def _make_scatter(T, D, P2, C):
    nc, ns, chunk = _sc_dims(T)
    half = chunk // 2
    mesh = plsc.VectorSubcoreMesh(core_axis_name="c", subcore_axis_name="s")

    q = chunk // 4

    @functools.partial(
        pl.kernel, mesh=mesh,
        out_type=[jax.ShapeDtypeStruct((P2, D), jnp.float32)],
        scratch_types=[
            pltpu.VMEM((q, D), jnp.float32),
            pltpu.VMEM((q, D), jnp.float32),
            pltpu.VMEM((chunk,), jnp.int32),
            pltpu.VMEM((chunk,), jnp.int32),
            pltpu.VMEM((chunk,), jnp.int32),
            pltpu.VMEM((chunk,), jnp.int32),
            pltpu.VMEM((q,), jnp.int32),
            pltpu.VMEM((q,), jnp.int32),
            pltpu.VMEM((q,), jnp.int32),
            pltpu.VMEM((q,), jnp.int32),
            pltpu.SemaphoreType.DMA,
            pltpu.SemaphoreType.DMA,
            pltpu.SemaphoreType.DMA,
            pltpu.SemaphoreType.DMA,
            pltpu.SemaphoreType.DMA,
            pltpu.SemaphoreType.DMA,
        ],
    )
    def scatter_k(xn_hbm, i0_hbm, i1_hbm, r0_hbm, r1_hbm, xg_hbm,
                  rows_a, rows_b, i0_v, i1_v, r0_v, r1_v,
                  pa0, pa1, pb0, pb1, la, lb, sa0, sa1, sb0, sb1):
        wid = lax.axis_index("s") * nc + lax.axis_index("c")
        base = wid * chunk
        pltpu.sync_copy(i0_hbm.at[pl.ds(base, chunk)], i0_v)
        pltpu.sync_copy(i1_hbm.at[pl.ds(base, chunk)], i1_v)
        pltpu.sync_copy(r0_hbm.at[pl.ds(base, chunk)], r0_v)
        pltpu.sync_copy(r1_hbm.at[pl.ds(base, chunk)], r1_v)

        rows = (rows_a, rows_b)
        idx = ((pa0, pa1), (pb0, pb1))
        lsem = (la, lb)
        ssem = ((sa0, sa1), (sb0, sb1))

        def fill(pb, iv, rv, t):
            for j in range(q // 16):
                sl = pl.ds(t * q + j * 16, 16)
                pb[pl.ds(j * 16, 16)] = iv[sl] * C + rv[sl]

        lds = [None, None]
        scs = [None, None]
        lds[0] = pltpu.async_copy(xn_hbm.at[pl.ds(base, q)], rows[0], la)
        lds[1] = pltpu.async_copy(xn_hbm.at[pl.ds(base + q, q)], rows[1], lb)
        for t in range(4):
            par = t % 2
            fill(idx[par][0], i0_v, r0_v, t)
            fill(idx[par][1], i1_v, r1_v, t)
            lds[par].wait()
            s0 = pltpu.async_copy(rows[par], xg_hbm.at[idx[par][0]],
                                  ssem[par][0])
            s1 = pltpu.async_copy(rows[par], xg_hbm.at[idx[par][1]],
                                  ssem[par][1])
            scs[par] = (s0, s1)
            if 1 <= t <= 2:
                opar = 1 - par
                scs[opar][0].wait()
                scs[opar][1].wait()
                lds[opar] = pltpu.async_copy(
                    xn_hbm.at[pl.ds(base + (t + 1) * q, q)], rows[opar],
                    lsem[opar])
        scs[0][0].wait()
        scs[0][1].wait()
        scs[1][0].wait()
        scs[1][1].wait()

    return scatter_k


def _make_gather(T, D, P2, C):
    nc, ns, chunk = _sc_dims(T)
    half = chunk // 2
    mesh = plsc.VectorSubcoreMesh(core_axis_name="c", subcore_axis_name="s")

    q = chunk // 4

    @functools.partial(
        pl.kernel, mesh=mesh,
        out_type=[
            jax.ShapeDtypeStruct((T, D), jnp.float32),
            jax.ShapeDtypeStruct((T, D), jnp.float32),
        ],
        scratch_types=[
            pltpu.VMEM((q, D), jnp.float32),
            pltpu.VMEM((q, D), jnp.float32),
            pltpu.VMEM((chunk,), jnp.int32),
            pltpu.VMEM((chunk,), jnp.int32),
            pltpu.VMEM((chunk,), jnp.int32),
            pltpu.VMEM((chunk,), jnp.int32),
            pltpu.VMEM((q,), jnp.int32),
            pltpu.VMEM((q,), jnp.int32),
            pltpu.SemaphoreType.DMA,
            pltpu.SemaphoreType.DMA,
            pltpu.SemaphoreType.DMA,
            pltpu.SemaphoreType.DMA,
        ],
    )
    def gather_k(yg_hbm, i0_hbm, i1_hbm, r0_hbm, r1_hbm,
                 r0out_hbm, r1out_hbm,
                 rows_a, rows_b, i0_v, i1_v, r0_v, r1_v,
                 pb0, pb1, ga, gb, wa, wb):
        wid = lax.axis_index("s") * nc + lax.axis_index("c")
        base = wid * chunk
        pltpu.sync_copy(i0_hbm.at[pl.ds(base, chunk)], i0_v)
        pltpu.sync_copy(i1_hbm.at[pl.ds(base, chunk)], i1_v)
        pltpu.sync_copy(r0_hbm.at[pl.ds(base, chunk)], r0_v)
        pltpu.sync_copy(r1_hbm.at[pl.ds(base, chunk)], r1_v)

        def fill(pb, iv, rv, t):
            for j in range(q // 16):
                sl = pl.ds(t * q + j * 16, 16)
                pb[pl.ds(j * 16, 16)] = iv[sl] * C + rv[sl]

        pw0 = pw1 = None
        for t in range(4):
            fill(pb0, i0_v, r0_v, t)
            fill(pb1, i1_v, r1_v, t)
            if pw0 is not None:
                pw0.wait()
                pw1.wait()
            g0 = pltpu.async_copy(yg_hbm.at[pb0], rows_a, ga)
            g1 = pltpu.async_copy(yg_hbm.at[pb1], rows_b, gb)
            g0.wait()
            pw0 = pltpu.async_copy(rows_a,
                                   r0out_hbm.at[pl.ds(base + t * q, q)], wa)
            g1.wait()
            pw1 = pltpu.async_copy(rows_b,
                                   r1out_hbm.at[pl.ds(base + t * q, q)], wb)
        pw0.wait()
        pw1.wait()

    return gather_k


def kernel(x, norm_w, Wr, W1s, b1s, W2s, b2s, Wgs, W1r, b1r, W2r, b2r, Wgr):
    B, S, D = x.shape
    E = Wr.shape[-1]
    H = W1r.shape[-1]
    T = B * S
    TM = _TM
    CAP = T
    CB = CAP // TM
    P2 = E * CAP + TM
    NB = (2 * T + E * TM) // TM
    BTA = 512

    xf = x.reshape(T, D)
    nw = norm_w.reshape(1, D)
    bf = jnp.bfloat16

    nt = T // BTA
    outs = pl.pallas_call(
        functools.partial(_route_body, nt, TM, NB, CB),
        grid=(nt,),
        in_specs=[
            pl.BlockSpec((BTA, D), lambda i: (i, 0)),
            pl.BlockSpec((1, D), lambda i: (0, 0)),
            pl.BlockSpec((D, E), lambda i: (0, 0)),
        ],
        out_specs=[
            pl.BlockSpec((BTA, D), lambda i: (i, 0)),
            pl.BlockSpec((BTA,), lambda i: (i,)),
            pl.BlockSpec((BTA,), lambda i: (i,)),
            pl.BlockSpec((BTA, 1), lambda i: (i, 0)),
            pl.BlockSpec((BTA, 1), lambda i: (i, 0)),
            pl.BlockSpec((BTA,), lambda i: (i,)),
            pl.BlockSpec((BTA,), lambda i: (i,)),
            pl.BlockSpec((1, NB), lambda i: (0, 0)),
            pl.BlockSpec((1, NB), lambda i: (0, 0)),
        ],
        out_shape=[
            jax.ShapeDtypeStruct((T, D), jnp.float32),
            jax.ShapeDtypeStruct((T,), jnp.int32),
            jax.ShapeDtypeStruct((T,), jnp.int32),
            jax.ShapeDtypeStruct((T, 1), jnp.float32),
            jax.ShapeDtypeStruct((T, 1), jnp.float32),
            jax.ShapeDtypeStruct((T,), jnp.int32),
            jax.ShapeDtypeStruct((T,), jnp.int32),
            jax.ShapeDtypeStruct((1, NB), jnp.int32),
            jax.ShapeDtypeStruct((1, NB), jnp.int32),
        ],
        scratch_shapes=[pltpu.VMEM((1, E), jnp.float32)],
    )(xf, nw, Wr)
    xn, i0f, i1f, w0, w1, r0f, r1f, blke, blkrow = outs

    (xg,) = _make_scatter(T, D, P2, CAP)(xn, i0f, i1f, r0f, r1f)

    w1r_b = W1r.astype(bf)
    w2r_b = W2r.astype(bf)
    wgr_b = Wgr.astype(bf)
    yg = pl.pallas_call(
        _ffn_body,
        grid_spec=pltpu.PrefetchScalarGridSpec(
            num_scalar_prefetch=2,
            grid=(NB,),
            in_specs=[
                pl.BlockSpec((TM, D), lambda b, be, br: (br[b], 0)),
                pl.BlockSpec((1, D, H), lambda b, be, br: (be[b], 0, 0)),
                pl.BlockSpec((H, H), lambda b, be, br: (0, 0)),
                pl.BlockSpec((1, H, D), lambda b, be, br: (be[b], 0, 0)),
                pl.BlockSpec((1, 1, H), lambda b, be, br: (be[b], 0, 0)),
                pl.BlockSpec((1, 1, D), lambda b, be, br: (be[b], 0, 0)),
            ],
            out_specs=pl.BlockSpec((TM, D), lambda b, be, br: (br[b], 0)),
        ),
        out_shape=jax.ShapeDtypeStruct((P2, D), jnp.float32),
    )(blke.reshape(NB), blkrow.reshape(NB), xg, w1r_b, wgr_b, w2r_b,
      b1r, b2r)

    rg0, rg1 = _make_gather(T, D, P2, CAP)(yg, i0f, i1f, r0f, r1f)

    BTC = 512
    out = pl.pallas_call(
        _combine_body,
        grid=(T // BTC,),
        in_specs=[
            pl.BlockSpec((BTC, D), lambda i: (i, 0)),
            pl.BlockSpec((1, D), lambda i: (0, 0)),
            pl.BlockSpec((BTC, D), lambda i: (i, 0)),
            pl.BlockSpec((BTC, D), lambda i: (i, 0)),
            pl.BlockSpec((BTC, 1), lambda i: (i, 0)),
            pl.BlockSpec((BTC, 1), lambda i: (i, 0)),
            pl.BlockSpec((D, H), lambda i: (0, 0)),
            pl.BlockSpec((H, H), lambda i: (0, 0)),
            pl.BlockSpec((H, D), lambda i: (0, 0)),
            pl.BlockSpec((1, H), lambda i: (0, 0)),
            pl.BlockSpec((1, D), lambda i: (0, 0)),
        ],
        out_specs=pl.BlockSpec((BTC, D), lambda i: (i, 0)),
        out_shape=jax.ShapeDtypeStruct((T, D), jnp.float32),
    )(xf, nw, rg0, rg1, w0, w1,
      W1s[0].astype(bf), Wgs.astype(bf), W2s[0].astype(bf),
      b1s[:, 0, :], b2s[:, 0, :])
    return out.reshape(B, S, D)


# speedup vs baseline: 1.1494x; 1.0415x over previous
import functools

import jax
import jax.numpy as jnp
from jax import lax
from jax.experimental import pallas as pl
from jax.experimental.pallas import tpu as pltpu
from jax.experimental.pallas import tpu_sc as plsc

_TM = 512


def _route_body(nt, tm, nb, cb, cap, x_ref, nw_ref, wr_ref,
                xn_ref, p0_ref, p1_ref, w0_ref, w1_ref,
                blke_ref, blkrow_ref, base_ref, ls_ref):
    i = pl.program_id(0)
    bt = x_ref.shape[0]
    e = wr_ref.shape[1]

    @pl.when(i == 0)
    def _():
        base_ref[...] = jnp.zeros_like(base_ref)
        ri = jax.lax.broadcasted_iota(jnp.int32, (bt, bt), 0)
        ci = jax.lax.broadcasted_iota(jnp.int32, (bt, bt), 1)
        ls_ref[...] = (ri > ci).astype(jnp.bfloat16)

    x = x_ref[...]
    xb = x.astype(jnp.bfloat16)
    ones = jnp.full((x.shape[1], 128), 1.0 / x.shape[1], jnp.bfloat16)
    ms = jnp.dot(xb * xb, ones, preferred_element_type=jnp.float32)[:, :1]
    xn = x * nw_ref[...] * jax.lax.rsqrt(ms + 1e-6)
    xn_ref[...] = xn

    aff = jnp.dot(xn, wr_ref[...], preferred_element_type=jnp.float32)
    iota = jax.lax.broadcasted_iota(jnp.int32, aff.shape, 1)
    m0 = jnp.max(aff, axis=-1, keepdims=True)
    i0 = jnp.min(jnp.where(aff == m0, iota, e), axis=-1, keepdims=True)
    aff1 = jnp.where(iota == i0, -jnp.inf, aff)
    m1 = jnp.max(aff1, axis=-1, keepdims=True)
    i1 = jnp.min(jnp.where(aff1 == m1, iota, e), axis=-1, keepdims=True)
    w0_ref[...] = m0
    w1_ref[...] = m1

    oh0 = (iota == i0).astype(jnp.float32)
    oh1 = (iota == i1).astype(jnp.float32)
    ohc = jnp.concatenate([oh0, oh1], axis=1).astype(jnp.bfloat16)
    cc = jnp.dot(ls_ref[...], ohc, preferred_element_type=jnp.float32)
    c0 = cc[:, :e]
    c1 = cc[:, e:]
    base = base_ref[...]
    s0 = jnp.sum(oh0, axis=0, keepdims=True)
    s1 = jnp.sum(oh1, axis=0, keepdims=True)
    r0 = jnp.sum(oh0 * (c0 + base), axis=-1, keepdims=True)
    r1 = jnp.sum(oh1 * (c1 + base + s0), axis=-1, keepdims=True)
    p0_ref[...] = (i0 * cap + r0.astype(jnp.int32))[:, 0]
    p1_ref[...] = (i1 * cap + r1.astype(jnp.int32))[:, 0]
    total = base + s0 + s1
    base_ref[...] = total

    nblk = jnp.floor((total + (tm - 1)) * (1.0 / tm))
    re = jax.lax.broadcasted_iota(jnp.int32, (e, e), 0)
    ce = jax.lax.broadcasted_iota(jnp.int32, (e, e), 1)
    lse = (re < ce).astype(jnp.float32)
    start = jnp.dot(nblk, lse, preferred_element_type=jnp.float32)
    ends = start + nblk
    ntot = jnp.sum(nblk, axis=-1, keepdims=True)
    bi = (jax.lax.broadcasted_iota(jnp.int32, (e, nb), 1)
          .astype(jnp.float32))
    eofb = jnp.sum((ends.reshape(e, 1) <= bi).astype(jnp.int32), axis=0,
                   keepdims=True)
    eofb = jnp.minimum(eofb, e - 1)
    ohb = (eofb == jax.lax.broadcasted_iota(jnp.int32, (e, nb), 0))
    startofb = jnp.sum(jnp.where(ohb, start.reshape(e, 1), 0.0), axis=0,
                       keepdims=True)
    bi0 = bi[0:1, :]
    row = (eofb * cb).astype(jnp.float32) + bi0 - startofb
    row = jnp.where(bi0 < ntot, row, float(e * cb))
    blke_ref[...] = eofb
    blkrow_ref[...] = row.astype(jnp.int32)


def _ffn_body(blke_ref, blkrow_ref, xg_ref, w1_ref, wg_ref, w2_ref,
              b1_ref, b2_ref, yg_ref):
    xb = xg_ref[...].astype(jnp.bfloat16)
    h = jnp.dot(xb, w1_ref[0], preferred_element_type=jnp.float32)
    h = h + b1_ref[0]
    g = jnp.dot(h.astype(jnp.bfloat16), wg_ref[...],
                preferred_element_type=jnp.float32)
    h = h * (g * (1.0 / (1.0 + jnp.exp(-g))))
    y = jnp.dot(h.astype(jnp.bfloat16), w2_ref[0],
                preferred_element_type=jnp.float32)
    yg_ref[...] = y + b2_ref[0]


def _combine_body(x_ref, nw_ref, r0_ref, r1_ref, w0_ref, w1_ref,
                  w1s_ref, wgs_ref, w2s_ref, b1s_ref, b2s_ref, out_ref):
    x = x_ref[...]
    xb = x.astype(jnp.bfloat16)
    ones = jnp.full((x.shape[1], 128), 1.0 / x.shape[1], jnp.bfloat16)
    ms = jnp.dot(xb * xb, ones, preferred_element_type=jnp.float32)[:, :1]
    xn = x * nw_ref[...] * jax.lax.rsqrt(ms + 1e-6)
    h = jnp.dot(xn.astype(jnp.bfloat16), w1s_ref[...],
                preferred_element_type=jnp.float32)
    h = h + b1s_ref[...]
    g = jnp.dot(h.astype(jnp.bfloat16), wgs_ref[...],
                preferred_element_type=jnp.float32)
    h = h * (g * (1.0 / (1.0 + jnp.exp(-g))))
    y = jnp.dot(h.astype(jnp.bfloat16), w2s_ref[...],
                preferred_element_type=jnp.float32)
    y = y + b2s_ref[...]
    out_ref[...] = (x + y
                    + w0_ref[...] * r0_ref[...] + w1_ref[...] * r1_ref[...])


def _sc_dims(T):
    info = plsc.get_sparse_core_info()
    nc, ns = info.num_cores, info.num_subcores
    return nc, ns, T // (nc * ns)


def _make_scatter(T, D, P2, C):
    nc, ns, chunk = _sc_dims(T)
    half = chunk // 2
    mesh = plsc.VectorSubcoreMesh(core_axis_name="c", subcore_axis_name="s")

    q = chunk // 4

    @functools.partial(
        pl.kernel, mesh=mesh,
        out_type=[jax.ShapeDtypeStruct((P2, D), jnp.float32)],
        scratch_types=[
            pltpu.VMEM((q, D), jnp.float32),
            pltpu.VMEM((q, D), jnp.float32),
            pltpu.VMEM((chunk,), jnp.int32),
            pltpu.VMEM((chunk,), jnp.int32),
            pltpu.VMEM((q,), jnp.int32),
            pltpu.VMEM((q,), jnp.int32),
            pltpu.VMEM((q,), jnp.int32),
            pltpu.VMEM((q,), jnp.int32),
            pltpu.SemaphoreType.DMA,
            pltpu.SemaphoreType.DMA,
            pltpu.SemaphoreType.DMA,
            pltpu.SemaphoreType.DMA,
            pltpu.SemaphoreType.DMA,
            pltpu.SemaphoreType.DMA,
        ],
    )
    def scatter_k(xn_hbm, p0_hbm, p1_hbm, xg_hbm,
                  rows_a, rows_b, p0_v, p1_v,
                  pa0, pa1, pb0, pb1, la, lb, sa0, sa1, sb0, sb1):
        wid = lax.axis_index("s") * nc + lax.axis_index("c")
        base = wid * chunk
        pltpu.sync_copy(p0_hbm.at[pl.ds(base, chunk)], p0_v)
        pltpu.sync_copy(p1_hbm.at[pl.ds(base, chunk)], p1_v)

        rows = (rows_a, rows_b)
        idx = ((pa0, pa1), (pb0, pb1))
        lsem = (la, lb)
        ssem = ((sa0, sa1), (sb0, sb1))

        def fill(pb, pv, t):
            for j in range(q // 16):
                pb[pl.ds(j * 16, 16)] = pv[pl.ds(t * q + j * 16, 16)]

        lds = [None, None]
        scs = [None, None]
        lds[0] = pltpu.async_copy(xn_hbm.at[pl.ds(base, q)], rows[0], la)
        lds[1] = pltpu.async_copy(xn_hbm.at[pl.ds(base + q, q)], rows[1], lb)
        for t in range(4):
            par = t % 2
            fill(idx[par][0], p0_v, t)
            fill(idx[par][1], p1_v, t)
            lds[par].wait()
            s0 = pltpu.async_copy(rows[par], xg_hbm.at[idx[par][0]],
                                  ssem[par][0])
            s1 = pltpu.async_copy(rows[par], xg_hbm.at[idx[par][1]],
                                  ssem[par][1])
            scs[par] = (s0, s1)
            if 1 <= t <= 2:
                opar = 1 - par
                scs[opar][0].wait()
                scs[opar][1].wait()
                lds[opar] = pltpu.async_copy(
                    xn_hbm.at[pl.ds(base + (t + 1) * q, q)], rows[opar],
                    lsem[opar])
        scs[0][0].wait()
        scs[0][1].wait()
        scs[1][0].wait()
        scs[1][1].wait()

    return scatter_k


def _make_gather(T, D, P2, C):
    nc, ns, chunk = _sc_dims(T)
    half = chunk // 2
    mesh = plsc.VectorSubcoreMesh(core_axis_name="c", subcore_axis_name="s")

    q = chunk // 4

    @functools.partial(
        pl.kernel, mesh=mesh,
        out_type=[
            jax.ShapeDtypeStruct((T, D), jnp.float32),
            jax.ShapeDtypeStruct((T, D), jnp.float32),
        ],
        scratch_types=[
            pltpu.VMEM((q, D), jnp.float32),
            pltpu.VMEM((q, D), jnp.float32),
            pltpu.VMEM((chunk,), jnp.int32),
            pltpu.VMEM((chunk,), jnp.int32),
            pltpu.SemaphoreType.DMA,
            pltpu.SemaphoreType.DMA,
            pltpu.SemaphoreType.DMA,
            pltpu.SemaphoreType.DMA,
        ],
    )
    def gather_k(yg_hbm, p0_hbm, p1_hbm,
                 r0out_hbm, r1out_hbm,
                 rows_a, rows_b, p0_v, p1_v,
                 ga, gb, wa, wb):
        wid = lax.axis_index("s") * nc + lax.axis_index("c")
        base = wid * chunk
        pltpu.sync_copy(p0_hbm.at[pl.ds(base, chunk)], p0_v)
        pltpu.sync_copy(p1_hbm.at[pl.ds(base, chunk)], p1_v)

        pw0 = pw1 = None
        for t in range(4):
            if pw0 is not None:
                pw0.wait()
                pw1.wait()
            g0 = pltpu.async_copy(yg_hbm.at[p0_v.at[pl.ds(t * q, q)]],
                                  rows_a, ga)
            g1 = pltpu.async_copy(yg_hbm.at[p1_v.at[pl.ds(t * q, q)]],
                                  rows_b, gb)
            g0.wait()
            pw0 = pltpu.async_copy(rows_a,
                                   r0out_hbm.at[pl.ds(base + t * q, q)], wa)
            g1.wait()
            pw1 = pltpu.async_copy(rows_b,
                                   r1out_hbm.at[pl.ds(base + t * q, q)], wb)
        pw0.wait()
        pw1.wait()

    return gather_k


def kernel(x, norm_w, Wr, W1s, b1s, W2s, b2s, Wgs, W1r, b1r, W2r, b2r, Wgr):
    B, S, D = x.shape
    E = Wr.shape[-1]
    H = W1r.shape[-1]
    T = B * S
    TM = _TM
    CAP = T
    CB = CAP // TM
    P2 = E * CAP + TM
    NB = (2 * T + E * TM) // TM
    BTA = 512

    xf = x.reshape(T, D)
    nw = norm_w.reshape(1, D)
    bf = jnp.bfloat16

    nt = T // BTA
    outs = pl.pallas_call(
        functools.partial(_route_body, nt, TM, NB, CB, CAP),
        grid=(nt,),
        in_specs=[
            pl.BlockSpec((BTA, D), lambda i: (i, 0)),
            pl.BlockSpec((1, D), lambda i: (0, 0)),
            pl.BlockSpec((D, E), lambda i: (0, 0)),
        ],
        out_specs=[
            pl.BlockSpec((BTA, D), lambda i: (i, 0)),
            pl.BlockSpec((BTA,), lambda i: (i,)),
            pl.BlockSpec((BTA,), lambda i: (i,)),
            pl.BlockSpec((BTA, 1), lambda i: (i, 0)),
            pl.BlockSpec((BTA, 1), lambda i: (i, 0)),
            pl.BlockSpec((1, NB), lambda i: (0, 0)),
            pl.BlockSpec((1, NB), lambda i: (0, 0)),
        ],
        out_shape=[
            jax.ShapeDtypeStruct((T, D), jnp.float32),
            jax.ShapeDtypeStruct((T,), jnp.int32),
            jax.ShapeDtypeStruct((T,), jnp.int32),
            jax.ShapeDtypeStruct((T, 1), jnp.float32),
            jax.ShapeDtypeStruct((T, 1), jnp.float32),
            jax.ShapeDtypeStruct((1, NB), jnp.int32),
            jax.ShapeDtypeStruct((1, NB), jnp.int32),
        ],
        scratch_shapes=[pltpu.VMEM((1, E), jnp.float32),
                        pltpu.VMEM((BTA, BTA), jnp.bfloat16)],
    )(xf, nw, Wr)
    xn, pos0, pos1, w0, w1, blke, blkrow = outs

    (xg,) = _make_scatter(T, D, P2, CAP)(xn, pos0, pos1)

    w1r_b = W1r.astype(bf)
    w2r_b = W2r.astype(bf)
    wgr_b = Wgr.astype(bf)
    yg = pl.pallas_call(
        _ffn_body,
        grid_spec=pltpu.PrefetchScalarGridSpec(
            num_scalar_prefetch=2,
            grid=(NB,),
            in_specs=[
                pl.BlockSpec((TM, D), lambda b, be, br: (br[b], 0)),
                pl.BlockSpec((1, D, H), lambda b, be, br: (be[b], 0, 0)),
                pl.BlockSpec((H, H), lambda b, be, br: (0, 0)),
                pl.BlockSpec((1, H, D), lambda b, be, br: (be[b], 0, 0)),
                pl.BlockSpec((1, 1, H), lambda b, be, br: (be[b], 0, 0)),
                pl.BlockSpec((1, 1, D), lambda b, be, br: (be[b], 0, 0)),
            ],
            out_specs=pl.BlockSpec((TM, D), lambda b, be, br: (br[b], 0)),
        ),
        out_shape=jax.ShapeDtypeStruct((P2, D), jnp.float32),
    )(blke.reshape(NB), blkrow.reshape(NB), xg, w1r_b, wgr_b, w2r_b,
      b1r, b2r)

    rg0, rg1 = _make_gather(T, D, P2, CAP)(yg, pos0, pos1)

    BTC = 512
    out = pl.pallas_call(
        _combine_body,
        grid=(T // BTC,),
        in_specs=[
            pl.BlockSpec((BTC, D), lambda i: (i, 0)),
            pl.BlockSpec((1, D), lambda i: (0, 0)),
            pl.BlockSpec((BTC, D), lambda i: (i, 0)),
            pl.BlockSpec((BTC, D), lambda i: (i, 0)),
            pl.BlockSpec((BTC, 1), lambda i: (i, 0)),
            pl.BlockSpec((BTC, 1), lambda i: (i, 0)),
            pl.BlockSpec((D, H), lambda i: (0, 0)),
            pl.BlockSpec((H, H), lambda i: (0, 0)),
            pl.BlockSpec((H, D), lambda i: (0, 0)),
            pl.BlockSpec((1, H), lambda i: (0, 0)),
            pl.BlockSpec((1, D), lambda i: (0, 0)),
        ],
        out_specs=pl.BlockSpec((BTC, D), lambda i: (i, 0)),
        out_shape=jax.ShapeDtypeStruct((T, D), jnp.float32),
    )(xf, nw, rg0, rg1, w0, w1,
      W1s[0].astype(bf), Wgs.astype(bf), W2s[0].astype(bf),
      b1s[:, 0, :], b2s[:, 0, :])
    return out.reshape(B, S, D)
